# baseline (device time: 10950 ns/iter reference)
import jax
import jax.numpy as jnp
from jax import lax
from jax.experimental import pallas as pl
from jax.experimental.pallas import tpu as pltpu

N_DEV = 8


def kernel(A, B):
    m, k = A.shape
    _, n = B.shape
    m_out = m // N_DEV

    def body(a_hbm, b_hbm, out_hbm, full_q, out_vmem, recv_q,
             scale_send, scale_recv, a_vmem, b_vmem,
             send_sems, recv_sems, s_send_sems, s_recv_sems,
             credit_sems, copy_sems):
        my = lax.axis_index("i")

        barrier_sem = pltpu.get_barrier_semaphore()
        pl.semaphore_signal(barrier_sem, inc=1)
        pl.semaphore_wait(barrier_sem, 1)

        a_copy = pltpu.make_async_copy(a_hbm, a_vmem, copy_sems.at[0])
        b_copy = pltpu.make_async_copy(b_hbm, b_vmem, copy_sems.at[1])
        a_copy.start()
        b_copy.start()

        for s in range(N_DEV - 1):
            peer = (my - s - 1) % N_DEV
            pl.semaphore_signal(
                credit_sems.at[s], inc=1,
                device_id=(peer,), device_id_type=pl.DeviceIdType.MESH,
            )

        a_copy.wait()
        b_copy.wait()

        part = jnp.dot(
            a_vmem[:, :].astype(jnp.bfloat16),
            b_vmem[:, :].astype(jnp.bfloat16),
            preferred_element_type=jnp.float32,
        )
        scale = jnp.max(jnp.abs(part)) * (1.0 / 127.0)
        full_q[:, :] = jnp.clip(
            jnp.round(part * (1.0 / scale)), -127.0, 127.0
        ).astype(jnp.int8)
        scale_send[:, :] = jnp.full((1, 128), scale, dtype=jnp.float32)

        sends = []
        for p in range(1, N_DEV):
            t = (my + p) % N_DEV
            pl.semaphore_wait(credit_sems.at[p - 1], 1)
            rdma = pltpu.make_async_remote_copy(
                src_ref=full_q.at[pl.ds(t * m_out, m_out), :],
                dst_ref=recv_q.at[p - 1],
                send_sem=send_sems.at[p - 1],
                recv_sem=recv_sems.at[p - 1],
                device_id=(t,),
                device_id_type=pl.DeviceIdType.MESH,
            )
            rdma.start()
            s_rdma = pltpu.make_async_remote_copy(
                src_ref=scale_send,
                dst_ref=scale_recv.at[p - 1],
                send_sem=s_send_sems.at[p - 1],
                recv_sem=s_recv_sems.at[p - 1],
                device_id=(t,),
                device_id_type=pl.DeviceIdType.MESH,
            )
            s_rdma.start()
            sends.append((rdma, s_rdma))

        for rdma, s_rdma in sends:
            rdma.wait_recv()
            s_rdma.wait_recv()
        acc = full_q[pl.ds(my * m_out, m_out), :].astype(jnp.float32) * scale
        for s in range(N_DEV - 1):
            acc = acc + recv_q[s, :, :].astype(jnp.float32) * scale_recv[s, 0, 0]
        out_vmem[:, :] = acc
        out_copy = pltpu.make_async_copy(out_vmem, out_hbm, copy_sems.at[2])
        out_copy.start()

        for rdma, s_rdma in sends:
            rdma.wait_send()
            s_rdma.wait_send()
        out_copy.wait()

    return pl.pallas_call(
        body,
        out_shape=jax.ShapeDtypeStruct((m_out, n), jnp.float32),
        in_specs=[
            pl.BlockSpec(memory_space=pl.ANY),
            pl.BlockSpec(memory_space=pl.ANY),
        ],
        out_specs=pl.BlockSpec(memory_space=pl.ANY),
        scratch_shapes=[
            pltpu.VMEM((m, n), jnp.int8),
            pltpu.VMEM((m_out, n), jnp.float32),
            pltpu.VMEM((N_DEV - 1, m_out, n), jnp.int8),
            pltpu.VMEM((1, 128), jnp.float32),
            pltpu.VMEM((N_DEV - 1, 1, 128), jnp.float32),
            pltpu.VMEM((m, k), jnp.float32),
            pltpu.VMEM((k, n), jnp.float32),
            pltpu.SemaphoreType.DMA((N_DEV - 1,)),
            pltpu.SemaphoreType.DMA((N_DEV - 1,)),
            pltpu.SemaphoreType.DMA((N_DEV - 1,)),
            pltpu.SemaphoreType.DMA((N_DEV - 1,)),
            pltpu.SemaphoreType.REGULAR((N_DEV - 1,)),
            pltpu.SemaphoreType.DMA((3,)),
        ],
        compiler_params=pltpu.CompilerParams(collective_id=0),
    )(A, B)


# device time: 10811 ns/iter; 1.0129x vs baseline; 1.0129x over previous
import jax
import jax.numpy as jnp
from jax import lax
from jax.experimental import pallas as pl
from jax.experimental.pallas import tpu as pltpu

N_DEV = 8


def kernel(A, B):
    m, k = A.shape
    _, n = B.shape
    m_out = m // N_DEV

    def body(a_hbm, b_hbm, out_ref, full_q, full_f32, recv_q,
             scale_send, scale_recv, a_vmem, b_vmem,
             send_sems, recv_sems, s_send_sems, s_recv_sems,
             credit_sems, copy_sems):
        my = lax.axis_index("i")

        barrier_sem = pltpu.get_barrier_semaphore()
        pl.semaphore_signal(barrier_sem, inc=1)
        pl.semaphore_wait(barrier_sem, 1)

        a_copy = pltpu.make_async_copy(a_hbm, a_vmem, copy_sems.at[0])
        b_copy = pltpu.make_async_copy(b_hbm, b_vmem, copy_sems.at[1])
        a_copy.start()
        b_copy.start()

        for s in range(N_DEV - 1):
            peer = (my - s - 1) % N_DEV
            pl.semaphore_signal(
                credit_sems.at[s], inc=1,
                device_id=(peer,), device_id_type=pl.DeviceIdType.MESH,
            )

        a_copy.wait()
        b_copy.wait()

        part = jnp.dot(
            a_vmem[:, :].astype(jnp.bfloat16),
            b_vmem[:, :].astype(jnp.bfloat16),
            preferred_element_type=jnp.float32,
        )
        full_f32[:, :] = part
        scale = jnp.max(jnp.abs(part)) * (1.0 / 127.0)
        full_q[:, :] = jnp.clip(
            jnp.round(part * (1.0 / scale)), -127.0, 127.0
        ).astype(jnp.int8)
        scale_send[:, :] = jnp.full((1, 128), scale, dtype=jnp.float32)

        sends = []
        for p in range(1, N_DEV):
            t = (my + p) % N_DEV
            pl.semaphore_wait(credit_sems.at[p - 1], 1)
            rdma = pltpu.make_async_remote_copy(
                src_ref=full_q.at[pl.ds(t * m_out, m_out), :],
                dst_ref=recv_q.at[p - 1],
                send_sem=send_sems.at[p - 1],
                recv_sem=recv_sems.at[p - 1],
                device_id=(t,),
                device_id_type=pl.DeviceIdType.MESH,
            )
            rdma.start()
            s_rdma = pltpu.make_async_remote_copy(
                src_ref=scale_send,
                dst_ref=scale_recv.at[p - 1],
                send_sem=s_send_sems.at[p - 1],
                recv_sem=s_recv_sems.at[p - 1],
                device_id=(t,),
                device_id_type=pl.DeviceIdType.MESH,
            )
            s_rdma.start()
            sends.append((rdma, s_rdma))

        for rdma, s_rdma in sends:
            rdma.wait_recv()
            s_rdma.wait_recv()
        acc = full_f32[pl.ds(my * m_out, m_out), :]
        for s in range(N_DEV - 1):
            acc = acc + recv_q[s, :, :].astype(jnp.float32) * scale_recv[s, 0, 0]
        out_ref[:, :] = acc

        for rdma, s_rdma in sends:
            rdma.wait_send()
            s_rdma.wait_send()

    return pl.pallas_call(
        body,
        out_shape=jax.ShapeDtypeStruct((m_out, n), jnp.float32),
        in_specs=[
            pl.BlockSpec(memory_space=pl.ANY),
            pl.BlockSpec(memory_space=pl.ANY),
        ],
        out_specs=pl.BlockSpec(memory_space=pltpu.VMEM),
        scratch_shapes=[
            pltpu.VMEM((m, n), jnp.int8),
            pltpu.VMEM((m, n), jnp.float32),
            pltpu.VMEM((N_DEV - 1, m_out, n), jnp.int8),
            pltpu.VMEM((1, 128), jnp.float32),
            pltpu.VMEM((N_DEV - 1, 1, 128), jnp.float32),
            pltpu.VMEM((m, k), jnp.float32),
            pltpu.VMEM((k, n), jnp.float32),
            pltpu.SemaphoreType.DMA((N_DEV - 1,)),
            pltpu.SemaphoreType.DMA((N_DEV - 1,)),
            pltpu.SemaphoreType.DMA((N_DEV - 1,)),
            pltpu.SemaphoreType.DMA((N_DEV - 1,)),
            pltpu.SemaphoreType.REGULAR((N_DEV - 1,)),
            pltpu.SemaphoreType.DMA((3,)),
        ],
        compiler_params=pltpu.CompilerParams(collective_id=0),
    )(A, B)
